# baseline (speedup 1.0000x reference)
import functools

import jax
import jax.numpy as jnp
from jax.experimental import pallas as pl
from jax.experimental.pallas import tpu as pltpu


def _round_up(x: int, m: int) -> int:
    return ((x + m - 1) // m) * m


def _linear_kernel(x_ref, w_hbm, b_ref, o_ref, wstage_ref, wbf_ref, wsem):
    n_pad, k_pad = wbf_ref.shape
    ch = wstage_ref.shape[1]
    nc = n_pad // ch

    def _wcopy(c):
        return pltpu.make_async_copy(
            w_hbm.at[pl.ds(c * ch, ch), :], wstage_ref.at[c % 2],
            wsem.at[c % 2])

    @pl.when(pl.program_id(0) == 0)
    def _cast_weight():
        for c in range(min(2, nc)):
            _wcopy(c).start()
        for c in range(nc):
            _wcopy(c).wait()
            wbf_ref[c * ch:(c + 1) * ch, :] = (
                wstage_ref[c % 2].astype(jnp.bfloat16))
            if c + 2 < nc:
                _wcopy(c + 2).start()

    acc = jax.lax.dot_general(
        x_ref[...].astype(jnp.bfloat16),
        wbf_ref[...],
        dimension_numbers=(((1,), (1,)), ((), ())),
        preferred_element_type=jnp.float32,
    )
    o_ref[...] = acc + b_ref[...]


@jax.jit
def _forward(x, weight, bias):
    B, K = x.shape
    N, K_w = weight.shape
    assert K == K_w, "weight in_features must match x feature dim"

    tm = min(512, _round_up(B, 8))
    B_pad = _round_up(B, tm)
    K_pad = _round_up(K, 128)
    N_pad = _round_up(N, 128)

    x_p = x if (B_pad == B and K_pad == K) else jnp.pad(
        x, ((0, B_pad - B), (0, K_pad - K)))
    w_p = weight if (N_pad == N and K_pad == K) else jnp.pad(
        weight, ((0, N_pad - N), (0, K_pad - K)))
    b_p = bias if N_pad == N else jnp.pad(bias, (0, N_pad - N))
    b2d = b_p.reshape(1, N_pad).astype(jnp.float32)

    gm = B_pad // tm
    flops = 2 * B_pad * K_pad * N_pad
    bytes_accessed = (4 * B_pad * K_pad + 4 * N_pad * K_pad
                      + 4 * N_pad + 4 * B_pad * N_pad)
    out_p = pl.pallas_call(
        _linear_kernel,
        out_shape=jax.ShapeDtypeStruct((B_pad, N_pad), jnp.float32),
        grid=(gm,),
        in_specs=[
            pl.BlockSpec((tm, K_pad), lambda j: (j, 0)),
            pl.BlockSpec(memory_space=pl.ANY),
            pl.BlockSpec((1, N_pad), lambda j: (0, 0)),
        ],
        out_specs=pl.BlockSpec((tm, N_pad), lambda j: (j, 0)),
        scratch_shapes=[
            pltpu.VMEM((2, N_pad // 4, K_pad), jnp.float32),
            pltpu.VMEM((N_pad, K_pad), jnp.bfloat16),
            pltpu.SemaphoreType.DMA((2,)),
        ],
        compiler_params=pltpu.CompilerParams(
            dimension_semantics=("arbitrary",),
            vmem_limit_bytes=64 * 1024 * 1024,
        ),
        cost_estimate=pl.CostEstimate(
            flops=flops, transcendentals=0, bytes_accessed=bytes_accessed),
    )(x_p, w_p, b2d)

    if B_pad == B and N_pad == N:
        return out_p
    return out_p[:B, :N]


def kernel(x, weight, bias):
    return _forward(x, weight, bias).astype(x.dtype)


# speedup vs baseline: 1.0341x; 1.0341x over previous
import functools

import jax
import jax.numpy as jnp
from jax.experimental import pallas as pl
from jax.experimental.pallas import tpu as pltpu


def _round_up(x: int, m: int) -> int:
    return ((x + m - 1) // m) * m


def _linear_kernel(x_ref, w_ref, b_ref, o_ref, wbf_ref):
    @pl.when(pl.program_id(0) == 0)
    def _cast_weight():
        wbf_ref[...] = w_ref[...].astype(jnp.bfloat16)

    acc = jax.lax.dot_general(
        x_ref[...].astype(jnp.bfloat16),
        wbf_ref[...],
        dimension_numbers=(((1,), (1,)), ((), ())),
        preferred_element_type=jnp.float32,
    )
    o_ref[...] = acc + b_ref[...]


@jax.jit
def _forward(x, weight, bias):
    B, K = x.shape
    N, K_w = weight.shape
    assert K == K_w, "weight in_features must match x feature dim"

    tm = min(512, _round_up(B, 8))
    B_pad = _round_up(B, tm)
    K_pad = _round_up(K, 128)
    N_pad = _round_up(N, 128)

    x_p = x if (B_pad == B and K_pad == K) else jnp.pad(
        x, ((0, B_pad - B), (0, K_pad - K)))
    w_p = weight if (N_pad == N and K_pad == K) else jnp.pad(
        weight, ((0, N_pad - N), (0, K_pad - K)))
    b_p = bias if N_pad == N else jnp.pad(bias, (0, N_pad - N))
    b2d = b_p.reshape(1, N_pad).astype(jnp.float32)

    gm = B_pad // tm
    flops = 2 * B_pad * K_pad * N_pad
    bytes_accessed = (4 * B_pad * K_pad + 4 * N_pad * K_pad
                      + 4 * N_pad + 4 * B_pad * N_pad)
    out_p = pl.pallas_call(
        _linear_kernel,
        out_shape=jax.ShapeDtypeStruct((B_pad, N_pad), jnp.float32),
        grid=(gm,),
        in_specs=[
            pl.BlockSpec((tm, K_pad), lambda j: (j, 0)),
            pl.BlockSpec((N_pad, K_pad), lambda j: (0, 0)),
            pl.BlockSpec((1, N_pad), lambda j: (0, 0)),
        ],
        out_specs=pl.BlockSpec((tm, N_pad), lambda j: (j, 0)),
        scratch_shapes=[pltpu.VMEM((N_pad, K_pad), jnp.bfloat16)],
        compiler_params=pltpu.CompilerParams(
            dimension_semantics=("arbitrary",),
            vmem_limit_bytes=64 * 1024 * 1024,
        ),
        cost_estimate=pl.CostEstimate(
            flops=flops, transcendentals=0, bytes_accessed=bytes_accessed),
    )(x_p, w_p, b2d)

    if B_pad == B and N_pad == N:
        return out_p
    return out_p[:B, :N]


def kernel(x, weight, bias):
    return _forward(x, weight, bias).astype(x.dtype)
